# baseline (speedup 1.0000x reference)
import jax
import jax.numpy as jnp
import numpy as np
from jax import lax
from jax.experimental import pallas as pl
from jax.experimental.pallas import tpu as pltpu

_N = 20000
_NP = 20480
_ROWS = _NP // 128
_NSEL = 1000
_NDET = 100
_OUT_ROWS = 104
_NMS_THRESH = 0.5
_SCORE_THRESH = 0.05
_CLIP = float(np.log(1000.0 / 16.0))
_IMG_H, _IMG_W = 600.0, 600.0


def _body(inp_ref, out_ref, sc_ref, b0_ref, b1_ref, b2_ref, b3_ref,
          k0_ref, k1_ref, k2_ref, k3_ref, kv_ref,
          s0_ref, s1_ref, s2_ref, s3_ref):
    x1 = inp_ref[0]
    y1 = inp_ref[1]
    x2 = inp_ref[2]
    y2 = inp_ref[3]
    w = x2 - x1
    h = y2 - y1
    cx = x1 + 0.5 * w
    cy = y1 + 0.5 * h
    dx = inp_ref[4] / 10.0
    dy = inp_ref[5] / 10.0
    dw = jnp.minimum(inp_ref[6] / 5.0, _CLIP)
    dh = jnp.minimum(inp_ref[7] / 5.0, _CLIP)
    pcx = dx * w + cx
    pcy = dy * h + cy
    pw = jnp.exp(dw) * w
    ph = jnp.exp(dh) * h
    b0_ref[...] = jnp.clip(pcx - 0.5 * pw, 0.0, _IMG_W)
    b1_ref[...] = jnp.clip(pcy - 0.5 * ph, 0.0, _IMG_H)
    b2_ref[...] = jnp.clip(pcx + 0.5 * pw, 0.0, _IMG_W)
    b3_ref[...] = jnp.clip(pcy + 0.5 * ph, 0.0, _IMG_H)

    l0 = inp_ref[8]
    l1 = inp_ref[9]
    lm = jnp.maximum(l0, l1)
    e0 = jnp.exp(l0 - lm)
    e1 = jnp.exp(l1 - lm)
    s = e1 / (e0 + e1)
    s = jnp.where(s > _SCORE_THRESH, s, 0.0)
    flat = (lax.broadcasted_iota(jnp.int32, (_ROWS, 128), 0) * 128
            + lax.broadcasted_iota(jnp.int32, (_ROWS, 128), 1))
    s = jnp.where(flat < _N, s, -1.0)
    sc_ref[...] = s

    kv_ref[...] = jnp.zeros((8, 128), jnp.float32)
    s0_ref[...] = jnp.zeros((8, 128), jnp.float32)
    s1_ref[...] = jnp.zeros((8, 128), jnp.float32)
    s2_ref[...] = jnp.zeros((8, 128), jnp.float32)
    s3_ref[...] = jnp.zeros((8, 128), jnp.float32)

    lanes = lax.broadcasted_iota(jnp.int32, (1, 128), 1)
    kidx = (lax.broadcasted_iota(jnp.int32, (8, 128), 0) * 128
            + lax.broadcasted_iota(jnp.int32, (8, 128), 1))
    lane5 = lax.broadcasted_iota(jnp.int32, (1, 5), 1)

    def load_box(idxm):
        sub = idxm // 128
        onehot = lanes == idxm % 128
        bx1 = jnp.sum(jnp.where(onehot, b0_ref[pl.ds(sub, 1), :], 0.0))
        by1 = jnp.sum(jnp.where(onehot, b1_ref[pl.ds(sub, 1), :], 0.0))
        bx2 = jnp.sum(jnp.where(onehot, b2_ref[pl.ds(sub, 1), :], 0.0))
        by2 = jnp.sum(jnp.where(onehot, b3_ref[pl.ds(sub, 1), :], 0.0))
        return bx1, by1, bx2, by2

    def extract(pm, pidx):
        s = sc_ref[...]
        sm = jnp.where((s < pm) | ((s == pm) & (flat > pidx)), s, -2.0)
        m = jnp.max(sm)
        idxm = jnp.min(jnp.where(sm == m, flat, jnp.int32(1 << 30)))
        return (m, idxm) + load_box(idxm)

    rows160 = lax.broadcasted_iota(jnp.int32, (_ROWS, 128), 0)
    rowi8 = lax.broadcasted_iota(jnp.int32, (8, 128), 0)
    cur = sc_ref[...]
    prows = []
    pflats = []
    for _k in range(8):
        cm = jnp.max(cur, axis=0, keepdims=True)
        rm = jnp.min(jnp.where(cur == cm, rows160, jnp.int32(1 << 30)),
                     axis=0, keepdims=True)
        prows.append(cm)
        pflats.append(rm * 128 + lanes)
        cur = jnp.where(rows160 == rm, -2.0, cur)
    pool0 = jnp.concatenate(prows, axis=0)
    fpool0 = jnp.concatenate(pflats, axis=0)

    def extract_pool(pool):
        m = jnp.max(pool)
        idxm = jnp.min(jnp.where(pool == m, fpool0, jnp.int32(1 << 30)))
        hit = (pool == m) & (fpool0 == idxm)
        exh = jnp.max(jnp.where(hit & (rowi8 == 7), 1, 0))
        npool = jnp.where(hit, -2.0, pool)
        return (m, idxm) + load_box(idxm) + (npool, exh)

    def nms_step(r, kc, m, bx1, by1, bx2, by2):
        kx1 = k0_ref[...]
        ky1 = k1_ref[...]
        kx2 = k2_ref[...]
        ky2 = k3_ref[...]
        kv = kv_ref[...]
        area_a = (kx2 - kx1) * (ky2 - ky1)
        area_b = (bx2 - bx1) * (by2 - by1)
        ltx = jnp.maximum(kx1, bx1)
        lty = jnp.maximum(ky1, by1)
        rbx = jnp.minimum(kx2, bx2)
        rby = jnp.minimum(ky2, by2)
        iw = jnp.clip(rbx - ltx, 0.0, None)
        ih = jnp.clip(rby - lty, 0.0, None)
        inter = iw * ih
        iou = inter / (area_a + area_b - inter + 1e-9)
        sup = (iou > _NMS_THRESH) & (kv > 0.5)
        nsup = jnp.max(jnp.where(sup, 1.0, 0.0))
        keep = nsup == 0.0

        at_k = (kidx == kc) & keep
        k0_ref[...] = jnp.where(at_k, bx1, kx1)
        k1_ref[...] = jnp.where(at_k, by1, ky1)
        k2_ref[...] = jnp.where(at_k, bx2, kx2)
        k3_ref[...] = jnp.where(at_k, by2, ky2)
        kv_ref[...] = jnp.where(at_k, 1.0, kv)

        p = jnp.where(keep & (kc < _NDET), kc, _NDET)
        row = jnp.where(lane5 == 0, bx1,
              jnp.where(lane5 == 1, by1,
              jnp.where(lane5 == 2, bx2,
              jnp.where(lane5 == 3, by2, m))))
        out_ref[pl.ds(p, 1), :] = row

        sq = r - kc
        at_s = (kidx == sq) & (~keep)
        s0_ref[...] = jnp.where(at_s, bx1, s0_ref[...])
        s1_ref[...] = jnp.where(at_s, by1, s1_ref[...])
        s2_ref[...] = jnp.where(at_s, bx2, s2_ref[...])
        s3_ref[...] = jnp.where(at_s, by2, s3_ref[...])

        return kc + jnp.where(keep, 1, 0)

    def body1(c):
        r, kc, m, idxm, bx1, by1, bx2, by2, pool, exh = c
        nm, nidx, nb1, nb2, nb3, nb4, npool, nexh = extract_pool(pool)
        kc2 = nms_step(r, kc, m, bx1, by1, bx2, by2)
        return (r + 1, kc2, nm, nidx, nb1, nb2, nb3, nb4, npool, nexh)

    fin1 = lax.while_loop(
        lambda c: (c[0] < _NSEL) & (c[1] < _NDET) & (c[9] == 0),
        body1, (jnp.int32(0), jnp.int32(0)) + extract_pool(pool0))
    r1, kc1, m1, idx1, c1, c2, c3, c4 = fin1[:8]

    def body2(c):
        r, kc, m, idxm, bx1, by1, bx2, by2 = c
        nxt = extract(m, idxm)
        kc2 = nms_step(r, kc, m, bx1, by1, bx2, by2)
        return (r + 1, kc2) + nxt

    fin2 = lax.while_loop(
        lambda c: (c[0] < _NSEL) & (c[1] < _NDET),
        body2, (r1, kc1, m1, idx1, c1, c2, c3, c4))
    kc = fin2[1]

    def fill(j):
        p = kc + j
        valid = p < _NDET
        onehot = lanes == j
        sx1 = jnp.sum(jnp.where(onehot, s0_ref[pl.ds(0, 1), :], 0.0))
        sy1 = jnp.sum(jnp.where(onehot, s1_ref[pl.ds(0, 1), :], 0.0))
        sx2 = jnp.sum(jnp.where(onehot, s2_ref[pl.ds(0, 1), :], 0.0))
        sy2 = jnp.sum(jnp.where(onehot, s3_ref[pl.ds(0, 1), :], 0.0))
        row = jnp.where(lane5 == 0, sx1,
              jnp.where(lane5 == 1, sy1,
              jnp.where(lane5 == 2, sx2,
              jnp.where(lane5 == 3, sy2, -1.0))))
        pw = jnp.where(valid, p, _NDET)
        out_ref[pl.ds(pw, 1), :] = row
        return j + 1

    lax.while_loop(lambda j: j < _NDET - kc, fill, jnp.int32(0))


def kernel(proposals, box_regression, logits):
    pad = _NP - _N
    P = jnp.pad(proposals.astype(jnp.float32), ((0, pad), (0, 0)))
    R = jnp.pad(box_regression.astype(jnp.float32), ((0, pad), (0, 0)))
    L = jnp.pad(logits.astype(jnp.float32), ((0, pad), (0, 0)))
    stk = jnp.concatenate([P, R, L], axis=1)
    inp = stk.T.reshape(10, _ROWS, 128)

    out = pl.pallas_call(
        _body,
        out_shape=jax.ShapeDtypeStruct((_OUT_ROWS, 5), jnp.float32),
        scratch_shapes=[
            pltpu.VMEM((_ROWS, 128), jnp.float32),
            pltpu.VMEM((_ROWS, 128), jnp.float32),
            pltpu.VMEM((_ROWS, 128), jnp.float32),
            pltpu.VMEM((_ROWS, 128), jnp.float32),
            pltpu.VMEM((_ROWS, 128), jnp.float32),
            pltpu.VMEM((8, 128), jnp.float32),
            pltpu.VMEM((8, 128), jnp.float32),
            pltpu.VMEM((8, 128), jnp.float32),
            pltpu.VMEM((8, 128), jnp.float32),
            pltpu.VMEM((8, 128), jnp.float32),
            pltpu.VMEM((8, 128), jnp.float32),
            pltpu.VMEM((8, 128), jnp.float32),
            pltpu.VMEM((8, 128), jnp.float32),
            pltpu.VMEM((8, 128), jnp.float32),
        ],
    )(inp)
    return out[:_NDET]


# speedup vs baseline: 1.1120x; 1.1120x over previous
import jax
import jax.numpy as jnp
import numpy as np
from jax import lax
from jax.experimental import pallas as pl
from jax.experimental.pallas import tpu as pltpu

_N = 20000
_NP = 20480
_ROWS = _NP // 128
_NSEL = 1000
_NDET = 100
_OUT_ROWS = 104
_NMS_THRESH = 0.5
_SCORE_THRESH = 0.05
_CLIP = float(np.log(1000.0 / 16.0))
_IMG_H, _IMG_W = 600.0, 600.0


def _body(inp_ref, out_ref, sc_ref, b0_ref, b1_ref, b2_ref, b3_ref,
          k0_ref, k1_ref, k2_ref, k3_ref, kv_ref,
          s0_ref, s1_ref, s2_ref, s3_ref):
    x1 = inp_ref[0]
    y1 = inp_ref[1]
    x2 = inp_ref[2]
    y2 = inp_ref[3]
    w = x2 - x1
    h = y2 - y1
    cx = x1 + 0.5 * w
    cy = y1 + 0.5 * h
    dx = inp_ref[4] / 10.0
    dy = inp_ref[5] / 10.0
    dw = jnp.minimum(inp_ref[6] / 5.0, _CLIP)
    dh = jnp.minimum(inp_ref[7] / 5.0, _CLIP)
    pcx = dx * w + cx
    pcy = dy * h + cy
    pw = jnp.exp(dw) * w
    ph = jnp.exp(dh) * h
    b0_ref[...] = jnp.clip(pcx - 0.5 * pw, 0.0, _IMG_W)
    b1_ref[...] = jnp.clip(pcy - 0.5 * ph, 0.0, _IMG_H)
    b2_ref[...] = jnp.clip(pcx + 0.5 * pw, 0.0, _IMG_W)
    b3_ref[...] = jnp.clip(pcy + 0.5 * ph, 0.0, _IMG_H)

    l0 = inp_ref[8]
    l1 = inp_ref[9]
    lm = jnp.maximum(l0, l1)
    e0 = jnp.exp(l0 - lm)
    e1 = jnp.exp(l1 - lm)
    s = e1 / (e0 + e1)
    s = jnp.where(s > _SCORE_THRESH, s, 0.0)
    flat = (lax.broadcasted_iota(jnp.int32, (_ROWS, 128), 0) * 128
            + lax.broadcasted_iota(jnp.int32, (_ROWS, 128), 1))
    s = jnp.where(flat < _N, s, -1.0)
    sc_ref[...] = s

    kv_ref[...] = jnp.zeros((8, 128), jnp.float32)
    s0_ref[...] = jnp.zeros((8, 128), jnp.float32)
    s1_ref[...] = jnp.zeros((8, 128), jnp.float32)
    s2_ref[...] = jnp.zeros((8, 128), jnp.float32)
    s3_ref[...] = jnp.zeros((8, 128), jnp.float32)

    lanes = lax.broadcasted_iota(jnp.int32, (1, 128), 1)
    kidx = (lax.broadcasted_iota(jnp.int32, (8, 128), 0) * 128
            + lax.broadcasted_iota(jnp.int32, (8, 128), 1))
    lane5 = lax.broadcasted_iota(jnp.int32, (1, 5), 1)

    def extract():
        s = sc_ref[...]
        m = jnp.max(s)
        idxm = jnp.min(jnp.where(s == m, flat, jnp.int32(1 << 30)))
        sub = idxm // 128
        lane = idxm % 128
        onehot = lanes == lane
        row_s = sc_ref[pl.ds(sub, 1), :]
        sc_ref[pl.ds(sub, 1), :] = jnp.where(onehot, -2.0, row_s)
        bx1 = jnp.sum(jnp.where(onehot, b0_ref[pl.ds(sub, 1), :], 0.0))
        by1 = jnp.sum(jnp.where(onehot, b1_ref[pl.ds(sub, 1), :], 0.0))
        bx2 = jnp.sum(jnp.where(onehot, b2_ref[pl.ds(sub, 1), :], 0.0))
        by2 = jnp.sum(jnp.where(onehot, b3_ref[pl.ds(sub, 1), :], 0.0))
        return m, bx1, by1, bx2, by2

    def body(carry):
        r, kc, m, bx1, by1, bx2, by2 = carry
        nxt = extract()

        kx1 = k0_ref[...]
        ky1 = k1_ref[...]
        kx2 = k2_ref[...]
        ky2 = k3_ref[...]
        kv = kv_ref[...]
        area_a = (kx2 - kx1) * (ky2 - ky1)
        area_b = (bx2 - bx1) * (by2 - by1)
        ltx = jnp.maximum(kx1, bx1)
        lty = jnp.maximum(ky1, by1)
        rbx = jnp.minimum(kx2, bx2)
        rby = jnp.minimum(ky2, by2)
        iw = jnp.clip(rbx - ltx, 0.0, None)
        ih = jnp.clip(rby - lty, 0.0, None)
        inter = iw * ih
        iou = inter / (area_a + area_b - inter + 1e-9)
        sup = (iou > _NMS_THRESH) & (kv > 0.5)
        nsup = jnp.max(jnp.where(sup, 1.0, 0.0))
        keep = nsup == 0.0

        at_k = (kidx == kc) & keep
        k0_ref[...] = jnp.where(at_k, bx1, kx1)
        k1_ref[...] = jnp.where(at_k, by1, ky1)
        k2_ref[...] = jnp.where(at_k, bx2, kx2)
        k3_ref[...] = jnp.where(at_k, by2, ky2)
        kv_ref[...] = jnp.where(at_k, 1.0, kv)

        p = jnp.where(keep & (kc < _NDET), kc, _NDET)
        row = jnp.where(lane5 == 0, bx1,
              jnp.where(lane5 == 1, by1,
              jnp.where(lane5 == 2, bx2,
              jnp.where(lane5 == 3, by2, m))))
        out_ref[pl.ds(p, 1), :] = row

        sq = r - kc
        at_s = (kidx == sq) & (~keep)
        s0_ref[...] = jnp.where(at_s, bx1, s0_ref[...])
        s1_ref[...] = jnp.where(at_s, by1, s1_ref[...])
        s2_ref[...] = jnp.where(at_s, bx2, s2_ref[...])
        s3_ref[...] = jnp.where(at_s, by2, s3_ref[...])

        return (r + 1, kc + jnp.where(keep, 1, 0)) + nxt

    fin = lax.while_loop(
        lambda c: (c[0] < _NSEL) & (c[1] < _NDET),
        body, (jnp.int32(0), jnp.int32(0)) + extract())
    kc = fin[1]

    def fill(j):
        p = kc + j
        valid = p < _NDET
        onehot = lanes == j
        sx1 = jnp.sum(jnp.where(onehot, s0_ref[pl.ds(0, 1), :], 0.0))
        sy1 = jnp.sum(jnp.where(onehot, s1_ref[pl.ds(0, 1), :], 0.0))
        sx2 = jnp.sum(jnp.where(onehot, s2_ref[pl.ds(0, 1), :], 0.0))
        sy2 = jnp.sum(jnp.where(onehot, s3_ref[pl.ds(0, 1), :], 0.0))
        row = jnp.where(lane5 == 0, sx1,
              jnp.where(lane5 == 1, sy1,
              jnp.where(lane5 == 2, sx2,
              jnp.where(lane5 == 3, sy2, -1.0))))
        pw = jnp.where(valid, p, _NDET)
        out_ref[pl.ds(pw, 1), :] = row
        return j + 1

    lax.while_loop(lambda j: j < _NDET - kc, fill, jnp.int32(0))


def kernel(proposals, box_regression, logits):
    pad = _NP - _N
    P = jnp.pad(proposals.astype(jnp.float32), ((0, pad), (0, 0)))
    R = jnp.pad(box_regression.astype(jnp.float32), ((0, pad), (0, 0)))
    L = jnp.pad(logits.astype(jnp.float32), ((0, pad), (0, 0)))
    stk = jnp.concatenate([P, R, L], axis=1)
    inp = stk.T.reshape(10, _ROWS, 128)

    out = pl.pallas_call(
        _body,
        out_shape=jax.ShapeDtypeStruct((_OUT_ROWS, 5), jnp.float32),
        scratch_shapes=[
            pltpu.VMEM((_ROWS, 128), jnp.float32),
            pltpu.VMEM((_ROWS, 128), jnp.float32),
            pltpu.VMEM((_ROWS, 128), jnp.float32),
            pltpu.VMEM((_ROWS, 128), jnp.float32),
            pltpu.VMEM((_ROWS, 128), jnp.float32),
            pltpu.VMEM((8, 128), jnp.float32),
            pltpu.VMEM((8, 128), jnp.float32),
            pltpu.VMEM((8, 128), jnp.float32),
            pltpu.VMEM((8, 128), jnp.float32),
            pltpu.VMEM((8, 128), jnp.float32),
            pltpu.VMEM((8, 128), jnp.float32),
            pltpu.VMEM((8, 128), jnp.float32),
            pltpu.VMEM((8, 128), jnp.float32),
            pltpu.VMEM((8, 128), jnp.float32),
        ],
    )(inp)
    return out[:_NDET]
